# Initial kernel scaffold; baseline (speedup 1.0000x reference)
import jax
import jax.numpy as jnp
from jax.experimental import pallas as pl


def kernel(ct, W1, b1, W2, b2, W3, b3, qm):
    raise NotImplementedError("write your pallas kernel here")


# speedup vs baseline: 1.1657x; 1.1657x over previous
import functools

import jax
import jax.numpy as jnp
from jax import lax
from jax.experimental import pallas as pl
from jax.experimental.pallas import tpu as pltpu
from jax.experimental.pallas import tpu_sc as plsc

CLS_DIM = 16384
HID = 1024
M = 8
K_SPARSE = 512
GAMMA_MAX = 0.2
MODULE_OUTPUT_CLIP_VAL = 100.0
ACTION_MEAN_CLIP_VAL = 1000.0

CHUNK = 1024
N_CHUNKS = CLS_DIM // CHUNK


def _h2_body(ct_ref, w1_ref, b1_ref, w2_ref, b2_ref, out_ref):
    c = pl.program_id(1)

    @pl.when(c == 0)
    def _():
        out_ref[0] = b1_ref[0]

    out_ref[0] += jnp.dot(ct_ref[0], w1_ref[0],
                          preferred_element_type=jnp.float32)

    @pl.when(c == N_CHUNKS - 1)
    def _():
        h1 = jnp.maximum(out_ref[0], 0.0)
        h2 = jnp.dot(h1, w2_ref[0], preferred_element_type=jnp.float32)
        out_ref[0] = jnp.maximum(h2 + b2_ref[0], 0.0)


def _dw_body(h2_ref, w3_ref, b3_ref, qm_ref, ct_ref, out_ref):
    pred = jnp.dot(h2_ref[0], w3_ref[0],
                   preferred_element_type=jnp.float32) + b3_ref[0, 0]
    pred = jnp.clip(pred, -MODULE_OUTPUT_CLIP_VAL, MODULE_OUTPUT_CLIP_VAL)
    z = qm_ref[0, 0] * ct_ref[0]
    gate = 1.0 / (1.0 + jnp.exp(-z))
    aw = gate * pred
    out_ref[0, 0] = jnp.clip(aw, -ACTION_MEAN_CLIP_VAL, ACTION_MEAN_CLIP_VAL)


def _thresh_body(dw_ref, out_ref):
    mag = jnp.abs(dw_ref[...])
    bits = lax.bitcast_convert_type(mag, jnp.uint32)

    def step(i, t):
        shift = (jnp.uint32(31) - i.astype(jnp.uint32))
        cand = t + jnp.left_shift(jnp.uint32(1), shift)
        ge = bits >= cand[:, :1]
        cnt = jnp.sum(ge.astype(jnp.int32), axis=1, keepdims=True)
        return jnp.where(cnt >= K_SPARSE, cand, t)

    t = lax.fori_loop(0, 32, step, jnp.zeros((M, 128), jnp.uint32))
    out_ref[...] = lax.bitcast_convert_type(t, jnp.float32)


def _make_sc_update():
    info = plsc.get_sparse_core_info()
    nc, ns, nl = info.num_cores, info.num_subcores, info.num_lanes
    nw = nc * ns
    span = CLS_DIM // nw
    decay = 1.0 - GAMMA_MAX

    mesh = plsc.VectorSubcoreMesh(core_axis_name="c", subcore_axis_name="s")

    @functools.partial(
        pl.kernel, mesh=mesh,
        out_type=jax.ShapeDtypeStruct((CLS_DIM,), jnp.float32),
        scratch_types=[
            pltpu.VMEM((M, span), jnp.float32),
            pltpu.VMEM((M, 128), jnp.float32),
            pltpu.VMEM((span,), jnp.float32),
            pltpu.VMEM((span,), jnp.float32),
        ],
    )
    def sc_update(dw_hbm, th_hbm, ct_hbm, out_hbm, x_v, th_v, ct_v, out_v):
        wid = lax.axis_index("s") * nc + lax.axis_index("c")
        base = wid * span
        for m in range(M):
            pltpu.sync_copy(dw_hbm.at[m, pl.ds(base, span)], x_v.at[m])
        pltpu.sync_copy(th_hbm, th_v)
        pltpu.sync_copy(ct_hbm.at[pl.ds(base, span)], ct_v)

        tvecs = [th_v[m, pl.ds(0, nl)] for m in range(M)]
        for j in range(span // nl):
            sl = pl.ds(j * nl, nl)
            acc = ct_v[sl] * decay
            for m in range(M):
                x = x_v[m, sl]
                keep = jnp.abs(x) >= tvecs[m]
                acc = acc + jnp.where(keep, x, 0.0)
            out_v[sl] = acc

        pltpu.sync_copy(out_v, out_hbm.at[pl.ds(base, span)])

    return sc_update


def kernel(ct, W1, b1, W2, b2, W3, b3, qm):
    ct3 = ct.reshape(N_CHUNKS, 1, CHUNK)
    b1r = b1.reshape(M, 1, HID)
    b2r = b2.reshape(M, 1, HID)
    b3r = b3.reshape(M, N_CHUNKS, 1, CHUNK)
    qmr = qm.reshape(M, N_CHUNKS, 1, CHUNK)

    h2 = pl.pallas_call(
        _h2_body,
        grid=(M, N_CHUNKS),
        in_specs=[
            pl.BlockSpec((1, 1, CHUNK), lambda m, c: (c, 0, 0)),
            pl.BlockSpec((1, CHUNK, HID), lambda m, c: (m, c, 0)),
            pl.BlockSpec((1, 1, HID), lambda m, c: (m, 0, 0)),
            pl.BlockSpec((1, HID, HID), lambda m, c: (m, 0, 0)),
            pl.BlockSpec((1, 1, HID), lambda m, c: (m, 0, 0)),
        ],
        out_specs=pl.BlockSpec((1, 1, HID), lambda m, c: (m, 0, 0)),
        out_shape=jax.ShapeDtypeStruct((M, 1, HID), jnp.float32),
        compiler_params=pltpu.CompilerParams(
            dimension_semantics=("arbitrary", "arbitrary")),
    )(ct3, W1, b1r, W2, b2r)

    dw4 = pl.pallas_call(
        _dw_body,
        grid=(M, N_CHUNKS),
        in_specs=[
            pl.BlockSpec((1, 1, HID), lambda m, c: (m, 0, 0)),
            pl.BlockSpec((1, HID, CHUNK), lambda m, c: (m, 0, c)),
            pl.BlockSpec((1, 1, 1, CHUNK), lambda m, c: (m, c, 0, 0)),
            pl.BlockSpec((1, 1, 1, CHUNK), lambda m, c: (m, c, 0, 0)),
            pl.BlockSpec((1, 1, CHUNK), lambda m, c: (c, 0, 0)),
        ],
        out_specs=pl.BlockSpec((1, 1, 1, CHUNK), lambda m, c: (m, c, 0, 0)),
        out_shape=jax.ShapeDtypeStruct((M, N_CHUNKS, 1, CHUNK), jnp.float32),
        compiler_params=pltpu.CompilerParams(
            dimension_semantics=("arbitrary", "arbitrary")),
    )(h2, W3, b3r, qmr, ct3)

    dw = dw4.reshape(M, CLS_DIM)

    th = pl.pallas_call(
        _thresh_body,
        out_shape=jax.ShapeDtypeStruct((M, 128), jnp.float32),
    )(dw)

    sc_update = _make_sc_update()
    return sc_update(dw, th, ct)


# speedup vs baseline: 1.2952x; 1.1111x over previous
import functools

import jax
import jax.numpy as jnp
from jax import lax
from jax.experimental import pallas as pl
from jax.experimental.pallas import tpu as pltpu
from jax.experimental.pallas import tpu_sc as plsc

CLS_DIM = 16384
HID = 1024
M = 8
K_SPARSE = 512
GAMMA_MAX = 0.2
MODULE_OUTPUT_CLIP_VAL = 100.0
ACTION_MEAN_CLIP_VAL = 1000.0

CHUNK = 2048
N_CHUNKS = CLS_DIM // CHUNK


def _h2_body(ct_ref, w1_ref, b1_ref, w2_ref, b2_ref, out_ref):
    c = pl.program_id(1)

    @pl.when(c == 0)
    def _():
        out_ref[0] = b1_ref[0]

    out_ref[0] += jnp.dot(ct_ref[0], w1_ref[0],
                          preferred_element_type=jnp.float32)

    @pl.when(c == N_CHUNKS - 1)
    def _():
        h1 = jnp.maximum(out_ref[0], 0.0)
        h2 = jnp.dot(h1, w2_ref[0], preferred_element_type=jnp.float32)
        out_ref[0] = jnp.maximum(h2 + b2_ref[0], 0.0)


def _dw_body(h2_ref, w3_ref, b3_ref, qm_ref, ct_ref, out_ref):
    pred = jnp.dot(h2_ref[0], w3_ref[0],
                   preferred_element_type=jnp.float32) + b3_ref[0, 0]
    pred = jnp.clip(pred, -MODULE_OUTPUT_CLIP_VAL, MODULE_OUTPUT_CLIP_VAL)
    z = qm_ref[0, 0] * ct_ref[0]
    gate = 1.0 / (1.0 + jnp.exp(-z))
    aw = gate * pred
    out_ref[0, 0] = jnp.clip(aw, -ACTION_MEAN_CLIP_VAL, ACTION_MEAN_CLIP_VAL)


def _thresh_body(dw_ref, out_ref):
    mag = jnp.abs(dw_ref[...])
    bits = lax.bitcast_convert_type(mag, jnp.uint32)

    def step(i, t):
        shift = (jnp.uint32(31) - i.astype(jnp.uint32))
        cand = t + jnp.left_shift(jnp.uint32(1), shift)
        ge = bits >= cand[:, :1]
        cnt = jnp.sum(ge.astype(jnp.int32), axis=1, keepdims=True)
        return jnp.where(cnt >= K_SPARSE, cand, t)

    t = lax.fori_loop(0, 32, step, jnp.zeros((M, 128), jnp.uint32))
    out_ref[...] = lax.bitcast_convert_type(t, jnp.float32)


def _make_sc_update():
    info = plsc.get_sparse_core_info()
    nc, ns, nl = info.num_cores, info.num_subcores, info.num_lanes
    nw = nc * ns
    span = CLS_DIM // nw
    decay = 1.0 - GAMMA_MAX

    mesh = plsc.VectorSubcoreMesh(core_axis_name="c", subcore_axis_name="s")

    @functools.partial(
        pl.kernel, mesh=mesh,
        out_type=jax.ShapeDtypeStruct((CLS_DIM,), jnp.float32),
        scratch_types=[
            pltpu.VMEM((M, span), jnp.float32),
            pltpu.VMEM((M, 128), jnp.float32),
            pltpu.VMEM((span,), jnp.float32),
            pltpu.VMEM((span,), jnp.float32),
        ],
    )
    def sc_update(dw_hbm, th_hbm, ct_hbm, out_hbm, x_v, th_v, ct_v, out_v):
        wid = lax.axis_index("s") * nc + lax.axis_index("c")
        base = wid * span
        for m in range(M):
            pltpu.sync_copy(dw_hbm.at[m, pl.ds(base, span)], x_v.at[m])
        pltpu.sync_copy(th_hbm, th_v)
        pltpu.sync_copy(ct_hbm.at[pl.ds(base, span)], ct_v)

        tvecs = [th_v[m, pl.ds(0, nl)] for m in range(M)]
        for j in range(span // nl):
            sl = pl.ds(j * nl, nl)
            acc = ct_v[sl] * decay
            for m in range(M):
                x = x_v[m, sl]
                keep = jnp.abs(x) >= tvecs[m]
                acc = acc + jnp.where(keep, x, 0.0)
            out_v[sl] = acc

        pltpu.sync_copy(out_v, out_hbm.at[pl.ds(base, span)])

    return sc_update


def kernel(ct, W1, b1, W2, b2, W3, b3, qm):
    ct3 = ct.reshape(N_CHUNKS, 1, CHUNK)
    b1r = b1.reshape(M, 1, HID)
    b2r = b2.reshape(M, 1, HID)
    b3r = b3.reshape(M, N_CHUNKS, 1, CHUNK)
    qmr = qm.reshape(M, N_CHUNKS, 1, CHUNK)

    h2 = pl.pallas_call(
        _h2_body,
        grid=(M, N_CHUNKS),
        in_specs=[
            pl.BlockSpec((1, 1, CHUNK), lambda m, c: (c, 0, 0)),
            pl.BlockSpec((1, CHUNK, HID), lambda m, c: (m, c, 0)),
            pl.BlockSpec((1, 1, HID), lambda m, c: (m, 0, 0)),
            pl.BlockSpec((1, HID, HID), lambda m, c: (m, 0, 0)),
            pl.BlockSpec((1, 1, HID), lambda m, c: (m, 0, 0)),
        ],
        out_specs=pl.BlockSpec((1, 1, HID), lambda m, c: (m, 0, 0)),
        out_shape=jax.ShapeDtypeStruct((M, 1, HID), jnp.float32),
        compiler_params=pltpu.CompilerParams(
            dimension_semantics=("arbitrary", "arbitrary")),
    )(ct3, W1, b1r, W2, b2r)

    dw4 = pl.pallas_call(
        _dw_body,
        grid=(M, N_CHUNKS),
        in_specs=[
            pl.BlockSpec((1, 1, HID), lambda m, c: (m, 0, 0)),
            pl.BlockSpec((1, HID, CHUNK), lambda m, c: (m, 0, c)),
            pl.BlockSpec((1, 1, 1, CHUNK), lambda m, c: (m, c, 0, 0)),
            pl.BlockSpec((1, 1, 1, CHUNK), lambda m, c: (m, c, 0, 0)),
            pl.BlockSpec((1, 1, CHUNK), lambda m, c: (c, 0, 0)),
        ],
        out_specs=pl.BlockSpec((1, 1, 1, CHUNK), lambda m, c: (m, c, 0, 0)),
        out_shape=jax.ShapeDtypeStruct((M, N_CHUNKS, 1, CHUNK), jnp.float32),
        compiler_params=pltpu.CompilerParams(
            dimension_semantics=("arbitrary", "arbitrary")),
    )(h2, W3, b3r, qmr, ct3)

    dw = dw4.reshape(M, CLS_DIM)

    th = pl.pallas_call(
        _thresh_body,
        out_shape=jax.ShapeDtypeStruct((M, 128), jnp.float32),
    )(dw)

    sc_update = _make_sc_update()
    return sc_update(dw, th, ct)
